# Initial kernel scaffold; baseline (speedup 1.0000x reference)
import functools

import jax
import jax.numpy as jnp
from jax import lax
from jax.experimental import pallas as pl
from jax.experimental.pallas import tpu as pltpu
from jax.experimental.pallas import tpu_sc as plsc

N = 10000
HID = 256
HALF = 128
E_REAL = 160000
E_TOT = E_REAL + N
NC = 2
NS = 16
L = 16
E_PAD = 172032
DEG_CHUNK = E_PAD // (NC * NS)
MP_CHUNK = 128
MP_PER_TILE = E_PAD // NS
MP_STEPS = MP_PER_TILE // MP_CHUNK
ROWS_PAD = 10016
ROWS_PER_TILE = ROWS_PAD // NS
LAST_ROWS = N - (NS - 1) * ROWS_PER_TILE
NEG = 0.01

_mesh = plsc.VectorSubcoreMesh(core_axis_name="c", subcore_axis_name="s")


@functools.partial(
    pl.kernel,
    out_type=(
        jax.ShapeDtypeStruct((NC * NS, ROWS_PAD), jnp.int32),
        jax.ShapeDtypeStruct((E_PAD,), jnp.float32),
    ),
    mesh=_mesh,
    scratch_types=[
        pltpu.VMEM((DEG_CHUNK,), jnp.int32),
        pltpu.VMEM((DEG_CHUNK,), jnp.float32),
        pltpu.VMEM((DEG_CHUNK,), jnp.float32),
        pltpu.VMEM((ROWS_PAD,), jnp.int32),
    ],
)
def _deg_dist(cols_hbm, attr_hbm, deg_hbm, dist_hbm, col_v, attr_v, dist_v, tbl_v):
    cid = lax.axis_index("c")
    sid = lax.axis_index("s")
    wid = sid * NC + cid
    base = wid * DEG_CHUNK

    pltpu.sync_copy(cols_hbm.at[pl.ds(base, DEG_CHUNK)], col_v)
    pltpu.sync_copy(attr_hbm.at[pl.ds(base, DEG_CHUNK)], attr_v)

    def zero_body(i, carry):
        tbl_v[pl.ds(i * L, L)] = jnp.zeros((L,), jnp.int32)
        return carry
    lax.fori_loop(0, ROWS_PAD // L, zero_body, 0)

    def hist_body(e, carry):
        c = col_v[e]
        tbl_v[c] = tbl_v[c] + 1
        return carry
    lax.fori_loop(0, DEG_CHUNK, hist_body, 0)

    def dist_body(i, carry):
        a = attr_v[pl.ds(i * L, L)]
        dist_v[pl.ds(i * L, L)] = jnp.exp(-(a * a))
        return carry
    lax.fori_loop(0, DEG_CHUNK // L, dist_body, 0)

    pltpu.sync_copy(tbl_v, deg_hbm.at[wid])
    pltpu.sync_copy(dist_v, dist_hbm.at[pl.ds(base, DEG_CHUNK)])


@functools.partial(
    pl.kernel,
    out_type=(
        jax.ShapeDtypeStruct((N, HALF), jnp.float32),
        jax.ShapeDtypeStruct((N, HALF), jnp.float32),
    ),
    mesh=_mesh,
    scratch_types=[
        pltpu.VMEM((MP_CHUNK,), jnp.int32),
        pltpu.VMEM((MP_CHUNK,), jnp.int32),
        pltpu.VMEM((MP_CHUNK,), jnp.float32),
        pltpu.VMEM((MP_CHUNK, HALF), jnp.float32),
        pltpu.VMEM_SHARED((ROWS_PAD, HALF), jnp.float32),
        pltpu.SemaphoreType.DMA,
    ],
)
def _msg_pass(ya_hbm, yb_hbm, rows_hbm, cols_hbm, dist_hbm,
              outa_hbm, outb_hbm, idx_v, col_v, dist_v, buf_v, acc, sem):
    cid = lax.axis_index("c")
    sid = lax.axis_index("s")

    def zbuf_body(r, carry):
        for k in range(HALF // L):
            buf_v[r, pl.ds(k * L, L)] = jnp.zeros((L,), jnp.float32)
        return carry
    lax.fori_loop(0, MP_CHUNK, zbuf_body, 0)
    r0 = sid * ROWS_PER_TILE
    for off, nrows in ((0, 128), (128, 128), (256, 128), (384, 128), (512, 114)):
        pltpu.sync_copy(buf_v.at[pl.ds(0, nrows)], acc.at[pl.ds(r0 + off, nrows)])
    plsc.subcore_barrier()

    def step(c, carry):
        off = sid * MP_PER_TILE + c * MP_CHUNK
        pltpu.sync_copy(rows_hbm.at[pl.ds(off, MP_CHUNK)], idx_v)
        pltpu.sync_copy(cols_hbm.at[pl.ds(off, MP_CHUNK)], col_v)
        pltpu.sync_copy(dist_hbm.at[pl.ds(off, MP_CHUNK)], dist_v)

        @pl.when(cid == 0)
        def _():
            pltpu.async_copy(ya_hbm.at[idx_v], buf_v, sem).wait()

        @pl.when(cid == 1)
        def _():
            pltpu.async_copy(yb_hbm.at[idx_v], buf_v, sem).wait()

        def scale_body(e, carry2):
            d = dist_v[e]
            for k in range(HALF // L):
                sl = pl.ds(k * L, L)
                buf_v[e, sl] = buf_v[e, sl] * d
            return carry2
        lax.fori_loop(0, MP_CHUNK, scale_body, 0)

        pltpu.sync_copy(buf_v, acc.at[col_v], add=True)
        return carry
    lax.fori_loop(0, MP_STEPS, step, 0)

    plsc.subcore_barrier()

    @pl.when(jnp.logical_and(cid == 0, sid < NS - 1))
    def _():
        pltpu.sync_copy(acc.at[pl.ds(r0, ROWS_PER_TILE)], outa_hbm.at[pl.ds(r0, ROWS_PER_TILE)])

    @pl.when(jnp.logical_and(cid == 0, sid == NS - 1))
    def _():
        pltpu.sync_copy(acc.at[pl.ds((NS - 1) * ROWS_PER_TILE, LAST_ROWS)],
                        outa_hbm.at[pl.ds((NS - 1) * ROWS_PER_TILE, LAST_ROWS)])

    @pl.when(jnp.logical_and(cid == 1, sid < NS - 1))
    def _():
        pltpu.sync_copy(acc.at[pl.ds(r0, ROWS_PER_TILE)], outb_hbm.at[pl.ds(r0, ROWS_PER_TILE)])

    @pl.when(jnp.logical_and(cid == 1, sid == NS - 1))
    def _():
        pltpu.sync_copy(acc.at[pl.ds((NS - 1) * ROWS_PER_TILE, LAST_ROWS)],
                        outb_hbm.at[pl.ds((NS - 1) * ROWS_PER_TILE, LAST_ROWS)])


_MB = 2000


def _dinv_of(deg_blk):
    return lax.rsqrt(jnp.sum(deg_blk.astype(jnp.float32), axis=0, keepdims=True))


def _mm0_body(x_ref, wt_ref, b_ref, deg_ref, ya_ref, yb_ref):
    dinv = _dinv_of(deg_ref[...])
    y = jnp.dot(x_ref[...], wt_ref[...], preferred_element_type=jnp.float32)
    y = (y + b_ref[...]) * dinv.T
    ya_ref[...] = y[:, :HALF]
    yb_ref[...] = y[:, HALF:]


def _mm1_body(sa_ref, sb_ref, deg_ref, wt_ref, b_ref, h1_ref, ya_ref, yb_ref):
    dinv = _dinv_of(deg_ref[...]).T
    s = jnp.concatenate([sa_ref[...], sb_ref[...]], axis=1) * dinv
    h1 = jnp.maximum(s, NEG * s)
    h1_ref[...] = h1
    y = jnp.dot(h1, wt_ref[...], preferred_element_type=jnp.float32)
    y = (y + b_ref[...]) * dinv
    ya_ref[...] = y[:, :HALF]
    yb_ref[...] = y[:, HALF:]


def _fin_body(x_ref, h1_ref, sa_ref, sb_ref, deg_ref, out_ref):
    dinv = _dinv_of(deg_ref[...]).T
    s = jnp.concatenate([sa_ref[...], sb_ref[...]], axis=1) * dinv
    h2 = jnp.maximum(s, NEG * s)
    out_ref[...] = (x_ref[...] + h1_ref[...] + h2) * (1.0 / 3.0)


def _full_spec(shape):
    return pl.BlockSpec(shape, lambda i: (0,) * len(shape))


def _row_spec(w):
    return pl.BlockSpec((_MB, w), lambda i: (i, 0))


_deg_spec = pl.BlockSpec((NC * NS, _MB), lambda i: (0, i))
_GRID = (N // _MB,)


def _mm0(x, wt, b, deg):
    return pl.pallas_call(
        _mm0_body,
        grid=_GRID,
        in_specs=[_row_spec(HID), _full_spec((HID, HID)), _full_spec((1, HID)), _deg_spec],
        out_specs=[_row_spec(HALF), _row_spec(HALF)],
        out_shape=[jax.ShapeDtypeStruct((N, HALF), jnp.float32)] * 2,
    )(x, wt, b, deg)


def _mm1(sa, sb, deg, wt, b):
    return pl.pallas_call(
        _mm1_body,
        grid=_GRID,
        in_specs=[_row_spec(HALF), _row_spec(HALF), _deg_spec,
                  _full_spec((HID, HID)), _full_spec((1, HID))],
        out_specs=[_row_spec(HID), _row_spec(HALF), _row_spec(HALF)],
        out_shape=[jax.ShapeDtypeStruct((N, HID), jnp.float32),
                   jax.ShapeDtypeStruct((N, HALF), jnp.float32),
                   jax.ShapeDtypeStruct((N, HALF), jnp.float32)],
    )(sa, sb, deg, wt, b)


def _fin(x, h1, sa, sb, deg):
    return pl.pallas_call(
        _fin_body,
        grid=_GRID,
        in_specs=[_row_spec(HID), _row_spec(HID), _row_spec(HALF), _row_spec(HALF), _deg_spec],
        out_specs=_row_spec(HID),
        out_shape=jax.ShapeDtypeStruct((N, HID), jnp.float32),
    )(x, h1, sa, sb, deg)


def kernel(poi_embs, edge_index, edge_attr, W0, b0, W1, b1):
    pad = E_PAD - E_TOT
    loops = jnp.arange(N, dtype=jnp.int32)
    rows = jnp.concatenate([edge_index[0], loops, jnp.zeros((pad,), jnp.int32)])
    cols = jnp.concatenate([edge_index[1], loops, jnp.full((pad,), N, jnp.int32)])
    attr = jnp.concatenate([edge_attr, jnp.zeros((N + pad,), jnp.float32)])

    deg, dist = _deg_dist(cols, attr)

    ya, yb = _mm0(poi_embs, W0.T, b0.reshape(1, HID), deg)
    s1a, s1b = _msg_pass(ya, yb, rows, cols, dist)
    h1, y1a, y1b = _mm1(s1a, s1b, deg, W1.T, b1.reshape(1, HID))
    s2a, s2b = _msg_pass(y1a, y1b, rows, cols, dist)
    return _fin(poi_embs, h1, s2a, s2b, deg)


# speedup vs baseline: 6.1877x; 6.1877x over previous
import functools

import jax
import jax.numpy as jnp
from jax import lax
from jax.experimental import pallas as pl
from jax.experimental.pallas import tpu as pltpu
from jax.experimental.pallas import tpu_sc as plsc

N = 10000
HID = 256
HALF = 128
E_REAL = 160000
E_TOT = E_REAL + N
NC = 2
NS = 16
L = 16
E_PAD = 172032
DEG_CHUNK = E_PAD // (NC * NS)
MP_CHUNK = 128
MP_PER_TILE = E_PAD // NS
MP_STEPS = MP_PER_TILE // MP_CHUNK
ROWS_PAD = 10112
ROWS_PER_TILE = ROWS_PAD // NS
LAST_ROWS = N - (NS - 1) * ROWS_PER_TILE
NEG = 0.01

_mesh = plsc.VectorSubcoreMesh(core_axis_name="c", subcore_axis_name="s")


@functools.partial(
    pl.kernel,
    out_type=(
        jax.ShapeDtypeStruct((NC * ROWS_PAD,), jnp.float32),
        jax.ShapeDtypeStruct((E_PAD,), jnp.float32),
    ),
    mesh=_mesh,
    scratch_types=[
        pltpu.VMEM((DEG_CHUNK,), jnp.int32),
        pltpu.VMEM((DEG_CHUNK,), jnp.float32),
        pltpu.VMEM((DEG_CHUNK,), jnp.float32),
        pltpu.VMEM((L,), jnp.float32),
        pltpu.VMEM((ROWS_PER_TILE,), jnp.float32),
        pltpu.VMEM_SHARED((ROWS_PAD,), jnp.float32),
    ],
)
def _deg_dist(cols_hbm, attr_hbm, deg_hbm, dist_hbm,
              col_v, attr_v, dist_v, ones_v, zero_v, deg_sh):
    cid = lax.axis_index("c")
    sid = lax.axis_index("s")
    wid = sid * NC + cid
    base = wid * DEG_CHUNK

    pltpu.sync_copy(cols_hbm.at[pl.ds(base, DEG_CHUNK)], col_v)
    pltpu.sync_copy(attr_hbm.at[pl.ds(base, DEG_CHUNK)], attr_v)

    ones_v[...] = jnp.ones((L,), jnp.float32)

    def zfill_body(i, carry):
        zero_v[pl.ds(jnp.minimum(i * L, ROWS_PER_TILE - L), L)] = jnp.zeros((L,), jnp.float32)
        return carry
    lax.fori_loop(0, (ROWS_PER_TILE + L - 1) // L, zfill_body, 0)

    pltpu.sync_copy(zero_v, deg_sh.at[pl.ds(sid * ROWS_PER_TILE, ROWS_PER_TILE)])
    plsc.subcore_barrier()

    def hist_body(i, carry):
        idx = col_v[pl.ds(i * L, L)]
        pltpu.sync_copy(ones_v, deg_sh.at[idx], add=True)
        return carry
    lax.fori_loop(0, DEG_CHUNK // L, hist_body, 0)

    def dist_body(i, carry):
        a = attr_v[pl.ds(i * L, L)]
        dist_v[pl.ds(i * L, L)] = jnp.exp(-(a * a))
        return carry
    lax.fori_loop(0, DEG_CHUNK // L, dist_body, 0)

    plsc.subcore_barrier()
    pltpu.sync_copy(deg_sh.at[pl.ds(sid * ROWS_PER_TILE, ROWS_PER_TILE)], zero_v)
    pltpu.sync_copy(zero_v, deg_hbm.at[pl.ds(cid * ROWS_PAD + sid * ROWS_PER_TILE, ROWS_PER_TILE)])
    pltpu.sync_copy(dist_v, dist_hbm.at[pl.ds(base, DEG_CHUNK)])


@functools.partial(
    pl.kernel,
    out_type=(
        jax.ShapeDtypeStruct((N, HALF), jnp.float32),
        jax.ShapeDtypeStruct((N, HALF), jnp.float32),
    ),
    mesh=_mesh,
    scratch_types=[
        pltpu.VMEM((MP_CHUNK,), jnp.int32),
        pltpu.VMEM((MP_CHUNK,), jnp.int32),
        pltpu.VMEM((MP_CHUNK,), jnp.float32),
        pltpu.VMEM((MP_CHUNK, HALF), jnp.float32),
        pltpu.VMEM_SHARED((ROWS_PAD, HALF), jnp.float32),
        pltpu.SemaphoreType.DMA,
    ],
)
def _msg_pass(ya_hbm, yb_hbm, rows_hbm, cols_hbm, dist_hbm,
              outa_hbm, outb_hbm, idx_v, col_v, dist_v, buf_v, acc, sem):
    cid = lax.axis_index("c")
    sid = lax.axis_index("s")

    def zbuf_body(r, carry):
        for k in range(HALF // L):
            buf_v[r, pl.ds(k * L, L)] = jnp.zeros((L,), jnp.float32)
        return carry
    lax.fori_loop(0, MP_CHUNK, zbuf_body, 0)
    r0 = sid * ROWS_PER_TILE
    for off, nrows in ((0, 128), (128, 128), (256, 128), (384, 128), (512, 120)):
        pltpu.sync_copy(buf_v.at[pl.ds(0, nrows)], acc.at[pl.ds(r0 + off, nrows)])
    plsc.subcore_barrier()

    def step(c, carry):
        off = sid * MP_PER_TILE + c * MP_CHUNK
        pltpu.sync_copy(rows_hbm.at[pl.ds(off, MP_CHUNK)], idx_v)
        pltpu.sync_copy(cols_hbm.at[pl.ds(off, MP_CHUNK)], col_v)
        pltpu.sync_copy(dist_hbm.at[pl.ds(off, MP_CHUNK)], dist_v)

        @pl.when(cid == 0)
        def _():
            pltpu.async_copy(ya_hbm.at[idx_v], buf_v, sem).wait()

        @pl.when(cid == 1)
        def _():
            pltpu.async_copy(yb_hbm.at[idx_v], buf_v, sem).wait()

        def scale_body(i, carry2):
            dv = dist_v[pl.ds(i * L, L)]
            for j in range(L):
                e = i * L + j
                d = dv[j]
                for k in range(HALF // L):
                    sl = pl.ds(k * L, L)
                    buf_v[e, sl] = buf_v[e, sl] * d
            return carry2
        lax.fori_loop(0, MP_CHUNK // L, scale_body, 0)

        pltpu.sync_copy(buf_v, acc.at[col_v], add=True)
        return carry
    lax.fori_loop(0, MP_STEPS, step, 0)

    plsc.subcore_barrier()

    def _copy_out(out_hbm):
        def _piece(off, nrows):
            pltpu.sync_copy(acc.at[pl.ds(r0 + off, nrows)], buf_v.at[pl.ds(0, nrows)])
            pltpu.sync_copy(buf_v.at[pl.ds(0, nrows)], out_hbm.at[pl.ds(r0 + off, nrows)])
        for off in (0, 128, 256, 384):
            _piece(off, 128)

        @pl.when(sid < NS - 1)
        def _():
            _piece(512, ROWS_PER_TILE - 512)

        @pl.when(sid == NS - 1)
        def _():
            _piece(512, LAST_ROWS - 512)

    @pl.when(cid == 0)
    def _():
        _copy_out(outa_hbm)

    @pl.when(cid == 1)
    def _():
        _copy_out(outb_hbm)


_MB = 2000


def _dinv_of(degt_blk):
    return lax.rsqrt(jnp.sum(degt_blk, axis=1, keepdims=True))


def _mm0_body(x_ref, wt_ref, b_ref, deg_ref, ya_ref, yb_ref):
    dinv = _dinv_of(deg_ref[...])
    y = jnp.dot(x_ref[...], wt_ref[...], preferred_element_type=jnp.float32)
    y = (y + b_ref[...]) * dinv
    ya_ref[...] = y[:, :HALF]
    yb_ref[...] = y[:, HALF:]


def _mm1_body(sa_ref, sb_ref, deg_ref, wt_ref, b_ref, h1_ref, ya_ref, yb_ref):
    dinv = _dinv_of(deg_ref[...])
    s = jnp.concatenate([sa_ref[...], sb_ref[...]], axis=1) * dinv
    h1 = jnp.maximum(s, NEG * s)
    h1_ref[...] = h1
    y = jnp.dot(h1, wt_ref[...], preferred_element_type=jnp.float32)
    y = (y + b_ref[...]) * dinv
    ya_ref[...] = y[:, :HALF]
    yb_ref[...] = y[:, HALF:]


def _fin_body(x_ref, h1_ref, sa_ref, sb_ref, deg_ref, out_ref):
    dinv = _dinv_of(deg_ref[...])
    s = jnp.concatenate([sa_ref[...], sb_ref[...]], axis=1) * dinv
    h2 = jnp.maximum(s, NEG * s)
    out_ref[...] = (x_ref[...] + h1_ref[...] + h2) * (1.0 / 3.0)


def _full_spec(shape):
    return pl.BlockSpec(shape, lambda i: (0,) * len(shape))


def _row_spec(w):
    return pl.BlockSpec((_MB, w), lambda i: (i, 0))


_deg_spec = pl.BlockSpec((_MB, NC), lambda i: (i, 0))
_GRID = (N // _MB,)


def _mm0(x, wt, b, deg):
    return pl.pallas_call(
        _mm0_body,
        grid=_GRID,
        in_specs=[_row_spec(HID), _full_spec((HID, HID)), _full_spec((1, HID)), _deg_spec],
        out_specs=[_row_spec(HALF), _row_spec(HALF)],
        out_shape=[jax.ShapeDtypeStruct((N, HALF), jnp.float32)] * 2,
    )(x, wt, b, deg)


def _mm1(sa, sb, deg, wt, b):
    return pl.pallas_call(
        _mm1_body,
        grid=_GRID,
        in_specs=[_row_spec(HALF), _row_spec(HALF), _deg_spec,
                  _full_spec((HID, HID)), _full_spec((1, HID))],
        out_specs=[_row_spec(HID), _row_spec(HALF), _row_spec(HALF)],
        out_shape=[jax.ShapeDtypeStruct((N, HID), jnp.float32),
                   jax.ShapeDtypeStruct((N, HALF), jnp.float32),
                   jax.ShapeDtypeStruct((N, HALF), jnp.float32)],
    )(sa, sb, deg, wt, b)


def _fin(x, h1, sa, sb, deg):
    return pl.pallas_call(
        _fin_body,
        grid=_GRID,
        in_specs=[_row_spec(HID), _row_spec(HID), _row_spec(HALF), _row_spec(HALF), _deg_spec],
        out_specs=_row_spec(HID),
        out_shape=jax.ShapeDtypeStruct((N, HID), jnp.float32),
    )(x, h1, sa, sb, deg)


def kernel(poi_embs, edge_index, edge_attr, W0, b0, W1, b1):
    pad = E_PAD - E_TOT
    loops = jnp.arange(N, dtype=jnp.int32)
    rows = jnp.concatenate([edge_index[0], loops, jnp.zeros((pad,), jnp.int32)])
    cols = jnp.concatenate([edge_index[1], loops, jnp.full((pad,), N, jnp.int32)])
    attr = jnp.concatenate([edge_attr, jnp.zeros((N + pad,), jnp.float32)])

    deg, dist = _deg_dist(cols, attr)
    degt = deg.reshape(NC, ROWS_PAD).T[:N]

    ya, yb = _mm0(poi_embs, W0.T, b0.reshape(1, HID), degt)
    s1a, s1b = _msg_pass(ya, yb, rows, cols, dist)
    h1, y1a, y1b = _mm1(s1a, s1b, degt, W1.T, b1.reshape(1, HID))
    s2a, s2b = _msg_pass(y1a, y1b, rows, cols, dist)
    return _fin(poi_embs, h1, s2a, s2b, degt)


# speedup vs baseline: 9.9014x; 1.6002x over previous
import functools

import jax
import jax.numpy as jnp
from jax import lax
from jax.experimental import pallas as pl
from jax.experimental.pallas import tpu as pltpu
from jax.experimental.pallas import tpu_sc as plsc

N = 10000
HID = 256
HALF = 128
E_REAL = 160000
E_TOT = E_REAL + N
NC = 2
NS = 16
L = 16
E_PAD = 172032
DEG_CHUNK = E_PAD // (NC * NS)
MP_CHUNK = 128
MP_PER_TILE = E_PAD // NS
MP_STEPS = MP_PER_TILE // MP_CHUNK
ROWS_PAD = 10112
ROWS_PER_TILE = ROWS_PAD // NS
LAST_ROWS = N - (NS - 1) * ROWS_PER_TILE
NEG = 0.01

_mesh = plsc.VectorSubcoreMesh(core_axis_name="c", subcore_axis_name="s")


@functools.partial(
    pl.kernel,
    out_type=(
        jax.ShapeDtypeStruct((NC * ROWS_PAD,), jnp.float32),
        jax.ShapeDtypeStruct((E_PAD,), jnp.float32),
    ),
    mesh=_mesh,
    scratch_types=[
        pltpu.VMEM((DEG_CHUNK,), jnp.int32),
        pltpu.VMEM((DEG_CHUNK,), jnp.float32),
        pltpu.VMEM((DEG_CHUNK,), jnp.float32),
        pltpu.VMEM((L,), jnp.float32),
        pltpu.VMEM((ROWS_PER_TILE,), jnp.float32),
        pltpu.VMEM_SHARED((ROWS_PAD,), jnp.float32),
    ],
)
def _deg_dist(cols_hbm, attr_hbm, deg_hbm, dist_hbm,
              col_v, attr_v, dist_v, ones_v, zero_v, deg_sh):
    cid = lax.axis_index("c")
    sid = lax.axis_index("s")
    wid = sid * NC + cid
    base = wid * DEG_CHUNK

    pltpu.sync_copy(cols_hbm.at[pl.ds(base, DEG_CHUNK)], col_v)
    pltpu.sync_copy(attr_hbm.at[pl.ds(base, DEG_CHUNK)], attr_v)

    ones_v[...] = jnp.ones((L,), jnp.float32)

    def zfill_body(i, carry):
        zero_v[pl.ds(jnp.minimum(i * L, ROWS_PER_TILE - L), L)] = jnp.zeros((L,), jnp.float32)
        return carry
    lax.fori_loop(0, (ROWS_PER_TILE + L - 1) // L, zfill_body, 0)

    pltpu.sync_copy(zero_v, deg_sh.at[pl.ds(sid * ROWS_PER_TILE, ROWS_PER_TILE)])
    plsc.subcore_barrier()

    def hist_body(i, carry):
        idx = col_v[pl.ds(i * L, L)]
        pltpu.sync_copy(ones_v, deg_sh.at[idx], add=True)
        return carry
    lax.fori_loop(0, DEG_CHUNK // L, hist_body, 0)

    def dist_body(i, carry):
        a = attr_v[pl.ds(i * L, L)]
        dist_v[pl.ds(i * L, L)] = jnp.exp(-(a * a))
        return carry
    lax.fori_loop(0, DEG_CHUNK // L, dist_body, 0)

    plsc.subcore_barrier()
    pltpu.sync_copy(deg_sh.at[pl.ds(sid * ROWS_PER_TILE, ROWS_PER_TILE)], zero_v)
    pltpu.sync_copy(zero_v, deg_hbm.at[pl.ds(cid * ROWS_PAD + sid * ROWS_PER_TILE, ROWS_PER_TILE)])
    pltpu.sync_copy(dist_v, dist_hbm.at[pl.ds(base, DEG_CHUNK)])


NPAIR = MP_STEPS // 2


@functools.partial(
    pl.kernel,
    out_type=(
        jax.ShapeDtypeStruct((N, HALF), jnp.float32),
        jax.ShapeDtypeStruct((N, HALF), jnp.float32),
    ),
    mesh=_mesh,
    scratch_types=[
        pltpu.VMEM((MP_CHUNK,), jnp.int32),
        pltpu.VMEM((MP_CHUNK,), jnp.int32),
        pltpu.VMEM((MP_CHUNK,), jnp.float32),
        pltpu.VMEM((MP_CHUNK,), jnp.int32),
        pltpu.VMEM((MP_CHUNK,), jnp.int32),
        pltpu.VMEM((MP_CHUNK,), jnp.float32),
        pltpu.VMEM((MP_CHUNK, HALF), jnp.float32),
        pltpu.VMEM((MP_CHUNK, HALF), jnp.float32),
        pltpu.VMEM_SHARED((ROWS_PAD, HALF), jnp.float32),
        pltpu.SemaphoreType.DMA,
        pltpu.SemaphoreType.DMA,
        pltpu.SemaphoreType.DMA,
        pltpu.SemaphoreType.DMA,
        pltpu.SemaphoreType.DMA,
        pltpu.SemaphoreType.DMA,
    ],
)
def _msg_pass(ya_hbm, yb_hbm, rows_hbm, cols_hbm, dist_hbm,
              outa_hbm, outb_hbm, idx0_v, col0_v, dist0_v, idx1_v, col1_v, dist1_v,
              rows0, rows1, acc, sem_g0, sem_g1, sem_s0, sem_s1, sem_m0, sem_m1):
    cid = lax.axis_index("c")
    sid = lax.axis_index("s")
    base = sid * MP_PER_TILE
    slots = ((idx0_v, col0_v, dist0_v, sem_g0, sem_s0, sem_m0, rows0),
             (idx1_v, col1_v, dist1_v, sem_g1, sem_s1, sem_m1, rows1))

    def zbuf_body(r, carry):
        for k in range(HALF // L):
            rows0[r, pl.ds(k * L, L)] = jnp.zeros((L,), jnp.float32)
        return carry
    lax.fori_loop(0, MP_CHUNK, zbuf_body, 0)
    r0 = sid * ROWS_PER_TILE
    for off, nrows in ((0, 128), (128, 128), (256, 128), (384, 128), (512, 120)):
        pltpu.sync_copy(rows0.at[pl.ds(0, nrows)], acc.at[pl.ds(r0 + off, nrows)])

    def meta_start(c, b):
        idx_v, col_v, dist_v, _, _, sem_m, _ = slots[b]
        off = base + c * MP_CHUNK
        pltpu.async_copy(rows_hbm.at[pl.ds(off, MP_CHUNK)], idx_v, sem_m)
        pltpu.async_copy(cols_hbm.at[pl.ds(off, MP_CHUNK)], col_v, sem_m)
        pltpu.async_copy(dist_hbm.at[pl.ds(off, MP_CHUNK)], dist_v, sem_m)

    def meta_drain(b):
        idx_v, col_v, dist_v, _, _, sem_m, _ = slots[b]
        pltpu.make_async_copy(rows_hbm.at[pl.ds(0, MP_CHUNK)], idx_v, sem_m).wait()
        pltpu.make_async_copy(cols_hbm.at[pl.ds(0, MP_CHUNK)], col_v, sem_m).wait()
        pltpu.make_async_copy(dist_hbm.at[pl.ds(0, MP_CHUNK)], dist_v, sem_m).wait()

    def gather_start(b):
        idx_v, _, _, sem_g, _, _, buf = slots[b]
        for c in range(NC):
            @pl.when(cid == c)
            def _():
                pltpu.async_copy((ya_hbm, yb_hbm)[c].at[idx_v], buf, sem_g)

    def gather_wait(b):
        idx_v, _, _, sem_g, _, _, buf = slots[b]
        for c in range(NC):
            @pl.when(cid == c)
            def _():
                pltpu.make_async_copy((ya_hbm, yb_hbm)[c].at[idx_v], buf, sem_g).wait()

    def scale(b):
        _, _, dist_v, _, _, _, buf = slots[b]

        def scale_body(i, carry):
            dv = dist_v[pl.ds(i * L, L)]
            for j in range(L):
                e = i * L + j
                d = dv[j]
                for k in range(HALF // L):
                    sl = pl.ds(k * L, L)
                    buf[e, sl] = buf[e, sl] * d
            return carry
        lax.fori_loop(0, MP_CHUNK // L, scale_body, 0)

    def scatter_start(b):
        _, col_v, _, _, sem_s, _, buf = slots[b]
        pltpu.async_copy(buf, acc.at[col_v], sem_s, add=True)

    def scatter_wait(b):
        _, col_v, _, _, sem_s, _, buf = slots[b]
        pltpu.make_async_copy(buf, acc.at[col_v], sem_s).wait()

    def phase(g, b):
        nb = 1 - b

        def _next_gather():
            meta_drain(nb)
            gather_start(nb)
        if b == 0:
            _next_gather()
        else:
            pl.when(g < NPAIR - 1)(_next_gather)

        gather_wait(b)
        scale(b)
        scatter_start(b)
        scatter_wait(b)

        @pl.when(g < NPAIR - 1)
        def _():
            meta_start(2 * g + b + 2, b)

    meta_start(0, 0)
    meta_start(1, 1)
    meta_drain(0)
    gather_start(0)
    plsc.subcore_barrier()

    def pair_body(g, carry):
        phase(g, 0)
        phase(g, 1)
        return carry
    lax.fori_loop(0, NPAIR, pair_body, 0)

    plsc.subcore_barrier()

    def _copy_out(out_hbm):
        def _piece(off, nrows):
            pltpu.sync_copy(acc.at[pl.ds(r0 + off, nrows)], rows0.at[pl.ds(0, nrows)])
            pltpu.sync_copy(rows0.at[pl.ds(0, nrows)], out_hbm.at[pl.ds(r0 + off, nrows)])
        for off in (0, 128, 256, 384):
            _piece(off, 128)

        @pl.when(sid < NS - 1)
        def _():
            _piece(512, ROWS_PER_TILE - 512)

        @pl.when(sid == NS - 1)
        def _():
            _piece(512, LAST_ROWS - 512)

    @pl.when(cid == 0)
    def _():
        _copy_out(outa_hbm)

    @pl.when(cid == 1)
    def _():
        _copy_out(outb_hbm)


_MB = 2000


def _dinv_of(degt_blk):
    return lax.rsqrt(jnp.sum(degt_blk, axis=1, keepdims=True))


def _mm0_body(x_ref, wt_ref, b_ref, deg_ref, ya_ref, yb_ref):
    dinv = _dinv_of(deg_ref[...])
    y = jnp.dot(x_ref[...], wt_ref[...], preferred_element_type=jnp.float32)
    y = (y + b_ref[...]) * dinv
    ya_ref[...] = y[:, :HALF]
    yb_ref[...] = y[:, HALF:]


def _mm1_body(sa_ref, sb_ref, deg_ref, wt_ref, b_ref, h1_ref, ya_ref, yb_ref):
    dinv = _dinv_of(deg_ref[...])
    s = jnp.concatenate([sa_ref[...], sb_ref[...]], axis=1) * dinv
    h1 = jnp.maximum(s, NEG * s)
    h1_ref[...] = h1
    y = jnp.dot(h1, wt_ref[...], preferred_element_type=jnp.float32)
    y = (y + b_ref[...]) * dinv
    ya_ref[...] = y[:, :HALF]
    yb_ref[...] = y[:, HALF:]


def _fin_body(x_ref, h1_ref, sa_ref, sb_ref, deg_ref, out_ref):
    dinv = _dinv_of(deg_ref[...])
    s = jnp.concatenate([sa_ref[...], sb_ref[...]], axis=1) * dinv
    h2 = jnp.maximum(s, NEG * s)
    out_ref[...] = (x_ref[...] + h1_ref[...] + h2) * (1.0 / 3.0)


def _full_spec(shape):
    return pl.BlockSpec(shape, lambda i: (0,) * len(shape))


def _row_spec(w):
    return pl.BlockSpec((_MB, w), lambda i: (i, 0))


_deg_spec = pl.BlockSpec((_MB, NC), lambda i: (i, 0))
_GRID = (N // _MB,)


def _mm0(x, wt, b, deg):
    return pl.pallas_call(
        _mm0_body,
        grid=_GRID,
        in_specs=[_row_spec(HID), _full_spec((HID, HID)), _full_spec((1, HID)), _deg_spec],
        out_specs=[_row_spec(HALF), _row_spec(HALF)],
        out_shape=[jax.ShapeDtypeStruct((N, HALF), jnp.float32)] * 2,
    )(x, wt, b, deg)


def _mm1(sa, sb, deg, wt, b):
    return pl.pallas_call(
        _mm1_body,
        grid=_GRID,
        in_specs=[_row_spec(HALF), _row_spec(HALF), _deg_spec,
                  _full_spec((HID, HID)), _full_spec((1, HID))],
        out_specs=[_row_spec(HID), _row_spec(HALF), _row_spec(HALF)],
        out_shape=[jax.ShapeDtypeStruct((N, HID), jnp.float32),
                   jax.ShapeDtypeStruct((N, HALF), jnp.float32),
                   jax.ShapeDtypeStruct((N, HALF), jnp.float32)],
    )(sa, sb, deg, wt, b)


def _fin(x, h1, sa, sb, deg):
    return pl.pallas_call(
        _fin_body,
        grid=_GRID,
        in_specs=[_row_spec(HID), _row_spec(HID), _row_spec(HALF), _row_spec(HALF), _deg_spec],
        out_specs=_row_spec(HID),
        out_shape=jax.ShapeDtypeStruct((N, HID), jnp.float32),
    )(x, h1, sa, sb, deg)


def kernel(poi_embs, edge_index, edge_attr, W0, b0, W1, b1):
    pad = E_PAD - E_TOT
    loops = jnp.arange(N, dtype=jnp.int32)
    rows = jnp.concatenate([edge_index[0], loops, jnp.zeros((pad,), jnp.int32)])
    cols = jnp.concatenate([edge_index[1], loops, jnp.full((pad,), N, jnp.int32)])
    attr = jnp.concatenate([edge_attr, jnp.zeros((N + pad,), jnp.float32)])

    deg, dist = _deg_dist(cols, attr)
    degt = deg.reshape(NC, ROWS_PAD).T[:N]

    ya, yb = _mm0(poi_embs, W0.T, b0.reshape(1, HID), degt)
    s1a, s1b = _msg_pass(ya, yb, rows, cols, dist)
    h1, y1a, y1b = _mm1(s1a, s1b, degt, W1.T, b1.reshape(1, HID))
    s2a, s2b = _msg_pass(y1a, y1b, rows, cols, dist)
    return _fin(poi_embs, h1, s2a, s2b, degt)
